# Initial kernel scaffold; baseline (speedup 1.0000x reference)
import jax
import jax.numpy as jnp
from jax.experimental import pallas as pl


def kernel(x, weight):
    raise NotImplementedError("write your pallas kernel here")


# speedup vs baseline: 4.0322x; 4.0322x over previous
import jax
import jax.numpy as jnp
from jax.experimental import pallas as pl
from jax.experimental.pallas import tpu as pltpu

_TM = 256
_VMEM_LIMIT_BYTES = 60 * 1024 * 1024


def _rotate_kernel(x_ref, w_ref, o_ref):
    o_ref[...] = jnp.dot(
        x_ref[...].astype(jnp.bfloat16),
        w_ref[...],
        preferred_element_type=jnp.float32,
    )


@jax.jit
def kernel(x, weight):
    H = weight.shape[0]
    lead = x.shape[:-1]
    x2d = x.astype(jnp.float32).reshape(-1, H)
    M = x2d.shape[0]

    tm = _TM if M % _TM == 0 else M
    pad = (-M) % tm
    if pad:
        x2d = jnp.pad(x2d, ((0, pad), (0, 0)))
    Mp = x2d.shape[0]

    w_bf16 = weight.astype(jnp.bfloat16)

    out = pl.pallas_call(
        _rotate_kernel,
        out_shape=jax.ShapeDtypeStruct((Mp, H), jnp.float32),
        grid=(Mp // tm,),
        in_specs=[
            pl.BlockSpec((tm, H), lambda i: (i, 0)),
            pl.BlockSpec((H, H), lambda i: (0, 0)),
        ],
        out_specs=pl.BlockSpec((tm, H), lambda i: (i, 0)),
        compiler_params=pltpu.CompilerParams(
            dimension_semantics=("parallel",),
            vmem_limit_bytes=_VMEM_LIMIT_BYTES,
        ),
        cost_estimate=pl.CostEstimate(
            flops=2 * Mp * H * H,
            bytes_accessed=4 * Mp * H + 2 * H * H + 4 * Mp * H,
            transcendentals=0,
        ),
    )(x2d, w_bf16)

    if pad:
        out = out[:M]
    return out.reshape(lead + (H,))
